# Initial kernel scaffold; baseline (speedup 1.0000x reference)
import functools

import jax
import jax.numpy as jnp
from jax import lax
from jax.experimental import pallas as pl
from jax.experimental.pallas import tpu as pltpu
from jax.experimental.pallas import tpu_sc as plsc

_NUM_CORES = 2
_NUM_SUBCORES = 16
_NUM_WORKERS = _NUM_CORES * _NUM_SUBCORES

_CHUNK = 128
_GROUP = 1024
_CHUNKS_PER_GROUP = _GROUP // _CHUNK


def _make_gather(num_rows: int, embed_dim: int):
  b_per_w = num_rows // _NUM_WORKERS
  groups_per_w = b_per_w // _GROUP
  mesh = plsc.VectorSubcoreMesh(core_axis_name="c", subcore_axis_name="s")

  @functools.partial(
      pl.kernel,
      out_type=jax.ShapeDtypeStruct((num_rows, embed_dim), jnp.float32),
      mesh=mesh,
      scratch_types=[
          pltpu.VMEM((b_per_w,), jnp.int32),
          pltpu.VMEM((_GROUP, embed_dim), jnp.float32),
          pltpu.SemaphoreType.DMA,
          pltpu.SemaphoreType.DMA,
      ],
  )
  def gather_kernel(table_hbm, idx_hbm, out_hbm, idx_v, rows_v, gsem, ssem):
    wid = lax.axis_index("s") * _NUM_CORES + lax.axis_index("c")
    base = wid * b_per_w
    pltpu.sync_copy(idx_hbm.at[pl.ds(base, b_per_w)], idx_v)

    def group_body(g, carry):
      goff = g * _GROUP
      copies = []
      for c in range(_CHUNKS_PER_GROUP):
        copies.append(
            pltpu.async_copy(
                table_hbm.at[idx_v.at[pl.ds(goff + c * _CHUNK, _CHUNK)]],
                rows_v.at[pl.ds(c * _CHUNK, _CHUNK)],
                gsem,
            )
        )
      for cp in copies:
        cp.wait()
      pltpu.async_copy(
          rows_v, out_hbm.at[pl.ds(base + goff, _GROUP)], ssem
      ).wait()
      return carry

    lax.fori_loop(0, groups_per_w, group_body, 0, unroll=False)

  return gather_kernel


def kernel(x, table):
  batch, hist = x.shape
  vocab, embed_dim = table.shape
  num_rows = batch * hist
  flat_idx = x.reshape(num_rows).astype(jnp.int32)
  out = _make_gather(num_rows, embed_dim)(table, flat_idx)
  return out.reshape(batch, hist, embed_dim)


# speedup vs baseline: 1.4776x; 1.4776x over previous
import functools

import jax
import jax.numpy as jnp
from jax import lax
from jax.experimental import pallas as pl
from jax.experimental.pallas import tpu as pltpu
from jax.experimental.pallas import tpu_sc as plsc

_NUM_CORES = 2
_NUM_SUBCORES = 16
_NUM_WORKERS = _NUM_CORES * _NUM_SUBCORES

_CHUNK = 128
_GROUP = 1024
_CHUNKS_PER_GROUP = _GROUP // _CHUNK


def _make_gather(num_rows: int, embed_dim: int):
  b_per_w = num_rows // _NUM_WORKERS
  groups_per_w = b_per_w // _GROUP
  mesh = plsc.VectorSubcoreMesh(core_axis_name="c", subcore_axis_name="s")

  @functools.partial(
      pl.kernel,
      out_type=jax.ShapeDtypeStruct((num_rows, embed_dim), jnp.float32),
      mesh=mesh,
      scratch_types=[
          pltpu.VMEM((b_per_w,), jnp.int32),
          pltpu.VMEM((_GROUP, embed_dim), jnp.float32),
          pltpu.SemaphoreType.DMA,
          pltpu.SemaphoreType.DMA,
      ],
      compiler_params=pltpu.CompilerParams(use_tc_tiling_on_sc=False),
  )
  def gather_kernel(table_hbm, idx_hbm, out_hbm, idx_v, rows_v, gsem, ssem):
    wid = lax.axis_index("s") * _NUM_CORES + lax.axis_index("c")
    base = wid * b_per_w
    pltpu.sync_copy(idx_hbm.at[pl.ds(base, b_per_w)], idx_v)

    def group_body(g, carry):
      goff = g * _GROUP
      copies = []
      for c in range(_CHUNKS_PER_GROUP):
        copies.append(
            pltpu.async_copy(
                table_hbm.at[idx_v.at[pl.ds(goff + c * _CHUNK, _CHUNK)]],
                rows_v.at[pl.ds(c * _CHUNK, _CHUNK)],
                gsem,
            )
        )
      for cp in copies:
        cp.wait()
      pltpu.async_copy(
          rows_v, out_hbm.at[pl.ds(base + goff, _GROUP)], ssem
      ).wait()
      return carry

    lax.fori_loop(0, groups_per_w, group_body, 0, unroll=False)

  return gather_kernel


def kernel(x, table):
  batch, hist = x.shape
  vocab, embed_dim = table.shape
  num_rows = batch * hist
  flat_idx = x.reshape(num_rows).astype(jnp.int32)
  out = _make_gather(num_rows, embed_dim)(table, flat_idx)
  return out.reshape(batch, hist, embed_dim)


# speedup vs baseline: 1.4782x; 1.0004x over previous
import functools

import jax
import jax.numpy as jnp
from jax import lax
from jax.experimental import pallas as pl
from jax.experimental.pallas import tpu as pltpu
from jax.experimental.pallas import tpu_sc as plsc

_NUM_CORES = 2
_NUM_SUBCORES = 16
_NUM_WORKERS = _NUM_CORES * _NUM_SUBCORES

_CHUNK = 1024
_GROUP = 1024
_CHUNKS_PER_GROUP = _GROUP // _CHUNK


def _make_gather(num_rows: int, embed_dim: int):
  b_per_w = num_rows // _NUM_WORKERS
  groups_per_w = b_per_w // _GROUP
  mesh = plsc.VectorSubcoreMesh(core_axis_name="c", subcore_axis_name="s")

  @functools.partial(
      pl.kernel,
      out_type=jax.ShapeDtypeStruct((num_rows, embed_dim), jnp.float32),
      mesh=mesh,
      scratch_types=[
          pltpu.VMEM((b_per_w,), jnp.int32),
          pltpu.VMEM((_GROUP, embed_dim), jnp.float32),
          pltpu.SemaphoreType.DMA,
          pltpu.SemaphoreType.DMA,
      ],
      compiler_params=pltpu.CompilerParams(use_tc_tiling_on_sc=False),
  )
  def gather_kernel(table_hbm, idx_hbm, out_hbm, idx_v, rows_v, gsem, ssem):
    wid = lax.axis_index("s") * _NUM_CORES + lax.axis_index("c")
    base = wid * b_per_w
    pltpu.sync_copy(idx_hbm.at[pl.ds(base, b_per_w)], idx_v)

    def group_body(g, carry):
      goff = g * _GROUP
      copies = []
      for c in range(_CHUNKS_PER_GROUP):
        copies.append(
            pltpu.async_copy(
                table_hbm.at[idx_v.at[pl.ds(goff + c * _CHUNK, _CHUNK)]],
                rows_v.at[pl.ds(c * _CHUNK, _CHUNK)],
                gsem,
            )
        )
      for cp in copies:
        cp.wait()
      pltpu.async_copy(
          rows_v, out_hbm.at[pl.ds(base + goff, _GROUP)], ssem
      ).wait()
      return carry

    lax.fori_loop(0, groups_per_w, group_body, 0, unroll=False)

  return gather_kernel


def kernel(x, table):
  batch, hist = x.shape
  vocab, embed_dim = table.shape
  num_rows = batch * hist
  flat_idx = x.reshape(num_rows).astype(jnp.int32)
  out = _make_gather(num_rows, embed_dim)(table, flat_idx)
  return out.reshape(batch, hist, embed_dim)


# speedup vs baseline: 1.4951x; 1.0114x over previous
import functools

import jax
import jax.numpy as jnp
from jax import lax
from jax.experimental import pallas as pl
from jax.experimental.pallas import tpu as pltpu
from jax.experimental.pallas import tpu_sc as plsc

_NUM_CORES = 2
_NUM_SUBCORES = 16
_NUM_WORKERS = _NUM_CORES * _NUM_SUBCORES

_GROUP = 1280


def _make_gather(num_rows: int, embed_dim: int):
  b_per_w = num_rows // _NUM_WORKERS
  num_groups = b_per_w // _GROUP
  assert num_groups % 2 == 0
  mesh = plsc.VectorSubcoreMesh(core_axis_name="c", subcore_axis_name="s")

  @functools.partial(
      pl.kernel,
      out_type=jax.ShapeDtypeStruct((num_rows, embed_dim), jnp.float32),
      mesh=mesh,
      scratch_types=[
          pltpu.VMEM((b_per_w,), jnp.int32),
          pltpu.VMEM((_GROUP, embed_dim), jnp.float32),
          pltpu.VMEM((_GROUP, embed_dim), jnp.float32),
          pltpu.SemaphoreType.DMA,
          pltpu.SemaphoreType.DMA,
      ],
      compiler_params=pltpu.CompilerParams(use_tc_tiling_on_sc=False),
  )
  def gather_kernel(table_hbm, idx_hbm, out_hbm, idx_v, rows0, rows1,
                    gsem, ssem):
    wid = lax.axis_index("s") * _NUM_CORES + lax.axis_index("c")
    base = wid * b_per_w
    bufs = (rows0, rows1)
    pltpu.sync_copy(idx_hbm.at[pl.ds(base, b_per_w)], idx_v)

    def fire_gather(g, buf):
      pltpu.async_copy(
          table_hbm.at[idx_v.at[pl.ds(g * _GROUP, _GROUP)]], buf, gsem
      )

    fire_gather(0, bufs[0])

    def pair_body(p, carry):
      for b in range(2):
        g = p * 2 + b
        buf, other = bufs[b], bufs[1 - b]
        pltpu.make_async_copy(
            table_hbm.at[idx_v.at[pl.ds(g * _GROUP, _GROUP)]], buf, gsem
        ).wait()
        @pl.when(g >= 1)
        def _():
          pltpu.make_async_copy(
              other, out_hbm.at[pl.ds(base + (g - 1) * _GROUP, _GROUP)], ssem
          ).wait()

        @pl.when(g + 1 < num_groups)
        def _():
          fire_gather(g + 1, other)

        pltpu.async_copy(
            buf, out_hbm.at[pl.ds(base + g * _GROUP, _GROUP)], ssem
        )
      return carry

    lax.fori_loop(0, num_groups // 2, pair_body, 0, unroll=False)
    pltpu.make_async_copy(
        bufs[1],
        out_hbm.at[pl.ds(base + (num_groups - 1) * _GROUP, _GROUP)],
        ssem,
    ).wait()

  return gather_kernel


def kernel(x, table):
  batch, hist = x.shape
  vocab, embed_dim = table.shape
  num_rows = batch * hist
  flat_idx = x.reshape(num_rows).astype(jnp.int32)
  out = _make_gather(num_rows, embed_dim)(table, flat_idx)
  return out.reshape(batch, hist, embed_dim)
